# baseline (speedup 1.0000x reference)
import functools

import jax
import jax.numpy as jnp
from jax import lax
from jax.experimental import pallas as pl
from jax.experimental.pallas import tpu as pltpu
from jax.experimental.pallas import tpu_sc as plsc

M, D, B = 100000, 64, 16384
NIDX = B * D
NELEM = M * D
NSC = 2
NTILE = 16
CHUNKS_PER_SC = 2
CHUNK_ROWS = M // (NSC * CHUNKS_PER_SC)
CHUNK = CHUNK_ROWS * D
PAD = 2048
SLAB = NIDX // NTILE
TSLICE = CHUNK // NTILE
QSLICE = 2000
NQ = TSLICE // QSLICE
BATCH = 4096
NBATCH = SLAB // BATCH

_mesh = plsc.VectorSubcoreMesh(core_axis_name="c", subcore_axis_name="s")


@functools.partial(
    pl.kernel,
    out_type=jax.ShapeDtypeStruct((NELEM,), jnp.float32),
    mesh=_mesh,
    scratch_types=[
        pltpu.VMEM_SHARED((CHUNK + PAD,), jnp.float32),
        pltpu.VMEM((BATCH,), jnp.int32),
        pltpu.VMEM((BATCH,), jnp.int32),
        pltpu.VMEM((BATCH,), jnp.int32),
        pltpu.VMEM((BATCH,), jnp.int32),
        pltpu.VMEM((BATCH,), jnp.float32),
        pltpu.VMEM((QSLICE,), jnp.float32),
        pltpu.VMEM((QSLICE,), jnp.float32),
        pltpu.VMEM((16,), jnp.int32),
        pltpu.SemaphoreType.DMA,
        pltpu.SemaphoreType.DMA,
        pltpu.SemaphoreType.DMA,
        pltpu.SemaphoreType.DMA,
    ],
)
def _scatter_add(x_hbm, idx_hbm, val_hbm, dim_hbm, out_hbm,
                 acc, ibuf0, ibuf1, fbuf0, fbuf1, vals, bn0, bn1, dimv,
                 sm0, sm1, sm2, sm3):
    c = lax.axis_index("c")
    s = lax.axis_index("s")
    iota = lax.iota(jnp.int32, 16)

    pltpu.sync_copy(val_hbm, vals)
    pltpu.sync_copy(dim_hbm, dimv)
    dim64 = dimv[pl.ds(0, 16)] * D

    x_flat = x_hbm
    idx_flat = idx_hbm
    out_flat = out_hbm

    for kk in range(CHUNKS_PER_SC):
        ebase = (c * CHUNKS_PER_SC + kk) * CHUNK
        coladj = [dim64 + iota + (j * 16 - ebase) for j in range(4)]
        dummy = [iota + (CHUNK + 128 * s + 16 * j) for j in range(4)]
        hbase = ebase + s * TSLICE
        abase = s * TSLICE

        @pl.loop(0, NQ // 2)
        def _(g):
            o0 = g * (2 * QSLICE)
            o1 = o0 + QSLICE
            d0 = pltpu.async_copy(x_flat.at[pl.ds(hbase + o0, QSLICE)], bn0, sm0)
            d1 = pltpu.async_copy(x_flat.at[pl.ds(hbase + o1, QSLICE)], bn1, sm1)
            d0.wait()
            a0 = pltpu.async_copy(bn0, acc.at[pl.ds(abase + o0, QSLICE)], sm2)
            d1.wait()
            a1 = pltpu.async_copy(bn1, acc.at[pl.ds(abase + o1, QSLICE)], sm3)
            a0.wait()
            a1.wait()

        plsc.subcore_barrier()

        @pl.loop(0, NBATCH // 2)
        def _(g):
            b0 = s * SLAB + (2 * g) * BATCH
            b1 = b0 + BATCH
            di0 = pltpu.async_copy(idx_flat.at[pl.ds(b0, BATCH)], ibuf0, sm0)
            di1 = pltpu.async_copy(idx_flat.at[pl.ds(b1, BATCH)], ibuf1, sm1)
            di0.wait()

            @plsc.parallel_loop(0, BATCH // 64, unroll=2)
            def _(t):
                for j in range(4):
                    v = ibuf0[pl.ds(t * 64 + j * 16, 16)]
                    l = v * 64 + coladj[j]
                    ok = plsc.bitcast(l, jnp.uint32) < jnp.uint32(CHUNK)
                    fbuf0[pl.ds(t * 64 + j * 16, 16)] = jnp.where(ok, l, dummy[j])

            sc0 = pltpu.async_copy(vals, acc.at[fbuf0], sm2, add=True)
            di1.wait()

            @plsc.parallel_loop(0, BATCH // 64, unroll=2)
            def _(t):
                for j in range(4):
                    v = ibuf1[pl.ds(t * 64 + j * 16, 16)]
                    l = v * 64 + coladj[j]
                    ok = plsc.bitcast(l, jnp.uint32) < jnp.uint32(CHUNK)
                    fbuf1[pl.ds(t * 64 + j * 16, 16)] = jnp.where(ok, l, dummy[j])

            sc1 = pltpu.async_copy(vals, acc.at[fbuf1], sm3, add=True)
            sc0.wait()
            sc1.wait()

        plsc.subcore_barrier()

        @pl.loop(0, NQ // 2)
        def _(g):
            o0 = g * (2 * QSLICE)
            o1 = o0 + QSLICE
            d0 = pltpu.async_copy(acc.at[pl.ds(abase + o0, QSLICE)], bn0, sm0)
            d1 = pltpu.async_copy(acc.at[pl.ds(abase + o1, QSLICE)], bn1, sm1)
            d0.wait()
            a0 = pltpu.async_copy(bn0, out_flat.at[pl.ds(hbase + o0, QSLICE)], sm2)
            d1.wait()
            a1 = pltpu.async_copy(bn1, out_flat.at[pl.ds(hbase + o1, QSLICE)], sm3)
            a0.wait()
            a1.wait()


def kernel(x, dim, index, value):
    idx_flat = lax.bitcast_convert_type(
        lax.bitcast_convert_type(index.astype(jnp.int32), jnp.float32)
        .reshape(-1),
        jnp.int32)
    vals = jnp.full((BATCH,), value, dtype=jnp.float32)
    dimarr = jnp.full((16,), dim, dtype=jnp.int32)
    out = _scatter_add(x.reshape(-1), idx_flat, vals, dimarr)
    return out.reshape(M, D)


# speedup vs baseline: 1.0020x; 1.0020x over previous
import functools

import jax
import jax.numpy as jnp
from jax import lax
from jax.experimental import pallas as pl
from jax.experimental.pallas import tpu as pltpu
from jax.experimental.pallas import tpu_sc as plsc

M, D, B = 100000, 64, 16384
NIDX = B * D
NELEM = M * D
NSC = 2
NTILE = 16
CHUNKS_PER_SC = 2
CHUNK_ROWS = M // (NSC * CHUNKS_PER_SC)
CHUNK = CHUNK_ROWS * D
PAD = 2048
SLAB = NIDX // NTILE
TSLICE = CHUNK // NTILE
QSLICE = 2000
NQ = TSLICE // QSLICE
BATCH = 4096
NBATCH = SLAB // BATCH

_mesh = plsc.VectorSubcoreMesh(core_axis_name="c", subcore_axis_name="s")


@functools.partial(
    pl.kernel,
    out_type=jax.ShapeDtypeStruct((NELEM,), jnp.float32),
    mesh=_mesh,
    scratch_types=[
        pltpu.VMEM_SHARED((CHUNK + PAD,), jnp.float32),
        pltpu.VMEM((BATCH,), jnp.int32),
        pltpu.VMEM((BATCH,), jnp.int32),
        pltpu.VMEM((BATCH,), jnp.int32),
        pltpu.VMEM((BATCH,), jnp.int32),
        pltpu.VMEM((BATCH,), jnp.float32),
        pltpu.VMEM((QSLICE,), jnp.float32),
        pltpu.VMEM((QSLICE,), jnp.float32),
        pltpu.VMEM((16,), jnp.int32),
        pltpu.SemaphoreType.DMA,
        pltpu.SemaphoreType.DMA,
        pltpu.SemaphoreType.DMA,
        pltpu.SemaphoreType.DMA,
    ],
)
def _scatter_add(x_hbm, idx_hbm, val_hbm, dim_hbm, out_hbm,
                 acc, ibuf0, ibuf1, fbuf0, fbuf1, vals, bn0, bn1, dimv,
                 sm0, sm1, sm2, sm3):
    c = lax.axis_index("c")
    s = lax.axis_index("s")
    iota = lax.iota(jnp.int32, 16)

    pltpu.sync_copy(val_hbm, vals)
    pltpu.sync_copy(dim_hbm, dimv)
    dim64 = dimv[pl.ds(0, 16)] * D

    x_flat = x_hbm
    idx_flat = idx_hbm
    out_flat = out_hbm

    for kk in range(CHUNKS_PER_SC):
        ebase = (c * CHUNKS_PER_SC + kk) * CHUNK
        coladj = [dim64 + iota + (j * 16 - ebase) for j in range(4)]
        dummy = [iota + (CHUNK + 128 * s + 16 * j) for j in range(4)]
        hbase = ebase + s * TSLICE
        abase = s * TSLICE

        @pl.loop(0, NQ // 2)
        def _(g):
            o0 = g * (2 * QSLICE)
            o1 = o0 + QSLICE
            d0 = pltpu.async_copy(x_flat.at[pl.ds(hbase + o0, QSLICE)], bn0, sm0)
            d1 = pltpu.async_copy(x_flat.at[pl.ds(hbase + o1, QSLICE)], bn1, sm1)
            d0.wait()
            a0 = pltpu.async_copy(bn0, acc.at[pl.ds(abase + o0, QSLICE)], sm2)
            d1.wait()
            a1 = pltpu.async_copy(bn1, acc.at[pl.ds(abase + o1, QSLICE)], sm3)
            a0.wait()
            a1.wait()

        plsc.subcore_barrier()

        @pl.loop(0, NBATCH // 2)
        def _(g):
            b0 = s * SLAB + (2 * g) * BATCH
            b1 = b0 + BATCH
            di0 = pltpu.async_copy(idx_flat.at[pl.ds(b0, BATCH)], ibuf0, sm0)
            di1 = pltpu.async_copy(idx_flat.at[pl.ds(b1, BATCH)], ibuf1, sm1)
            di0.wait()

            @plsc.parallel_loop(0, BATCH // 64, unroll=2)
            def _(t):
                for j in range(4):
                    v = ibuf0[pl.ds(t * 64 + j * 16, 16)]
                    l = v * 64 + coladj[j]
                    ok = plsc.bitcast(l, jnp.uint32) < jnp.uint32(CHUNK)
                    fbuf0[pl.ds(t * 64 + j * 16, 16)] = jnp.where(ok, l, dummy[j])

            sc0 = pltpu.async_copy(vals, acc.at[fbuf0], sm2, add=True)
            di1.wait()

            @plsc.parallel_loop(0, BATCH // 64, unroll=2)
            def _(t):
                for j in range(4):
                    v = ibuf1[pl.ds(t * 64 + j * 16, 16)]
                    l = v * 64 + coladj[j]
                    ok = plsc.bitcast(l, jnp.uint32) < jnp.uint32(CHUNK)
                    fbuf1[pl.ds(t * 64 + j * 16, 16)] = jnp.where(ok, l, dummy[j])

            sc1 = pltpu.async_copy(vals, acc.at[fbuf1], sm3, add=True)
            sc0.wait()
            sc1.wait()

        plsc.subcore_barrier()

        @pl.loop(0, NQ // 2)
        def _(g):
            o0 = g * (2 * QSLICE)
            o1 = o0 + QSLICE
            d0 = pltpu.async_copy(acc.at[pl.ds(abase + o0, QSLICE)], bn0, sm0)
            d1 = pltpu.async_copy(acc.at[pl.ds(abase + o1, QSLICE)], bn1, sm1)
            d0.wait()
            a0 = pltpu.async_copy(bn0, out_flat.at[pl.ds(hbase + o0, QSLICE)], sm2)
            d1.wait()
            a1 = pltpu.async_copy(bn1, out_flat.at[pl.ds(hbase + o1, QSLICE)], sm3)
            a0.wait()
            a1.wait()


def kernel(x, dim, index, value):
    idx_flat = index.astype(jnp.int32).reshape(-1)
    vals = jnp.full((BATCH,), value, dtype=jnp.float32)
    dimarr = jnp.full((16,), dim, dtype=jnp.int32)
    out = _scatter_add(x.reshape(-1), idx_flat, vals, dimarr)
    return out.reshape(M, D)


# speedup vs baseline: 1.0789x; 1.0767x over previous
import functools

import jax
import jax.numpy as jnp
from jax import lax
from jax.experimental import pallas as pl
from jax.experimental.pallas import tpu as pltpu
from jax.experimental.pallas import tpu_sc as plsc

M, D, B = 100000, 64, 16384
NIDX = B * D
NELEM = M * D
NSC = 2
NTILE = 16
CHUNKS_PER_SC = 2
CHUNK_ROWS = M // (NSC * CHUNKS_PER_SC)
CHUNK = CHUNK_ROWS * D
PAD = 2048
SLAB = NIDX // NTILE
TSLICE = CHUNK // NTILE
QSLICE = 2000
NQ = TSLICE // QSLICE
BATCH = 4096
NBATCH = SLAB // BATCH

_mesh = plsc.VectorSubcoreMesh(core_axis_name="c", subcore_axis_name="s")


@functools.partial(
    pl.kernel,
    out_type=jax.ShapeDtypeStruct((NELEM,), jnp.float32),
    mesh=_mesh,
    scratch_types=[
        pltpu.VMEM_SHARED((CHUNK + PAD,), jnp.float32),
        pltpu.VMEM((BATCH,), jnp.int32),
        pltpu.VMEM((BATCH,), jnp.int32),
        pltpu.VMEM((BATCH,), jnp.int32),
        pltpu.VMEM((BATCH,), jnp.int32),
        pltpu.VMEM((BATCH,), jnp.float32),
        pltpu.VMEM((QSLICE,), jnp.float32),
        pltpu.VMEM((QSLICE,), jnp.float32),
        pltpu.VMEM((16,), jnp.int32),
        pltpu.SemaphoreType.DMA,
        pltpu.SemaphoreType.DMA,
        pltpu.SemaphoreType.DMA,
        pltpu.SemaphoreType.DMA,
    ],
)
def _scatter_add(x_hbm, idx_hbm, val_hbm, dim_hbm, out_hbm,
                 acc, ibuf0, ibuf1, fbuf0, fbuf1, vals, bn0, bn1, dimv,
                 sm0, sm1, sm2, sm3):
    c = lax.axis_index("c")
    s = lax.axis_index("s")
    iota = lax.iota(jnp.int32, 16)

    pltpu.sync_copy(val_hbm, vals)
    pltpu.sync_copy(dim_hbm, dimv)
    dim64 = dimv[pl.ds(0, 16)] * D

    x_flat = x_hbm
    idx_flat = idx_hbm
    out_flat = out_hbm

    for kk in range(CHUNKS_PER_SC):
        ebase = (c * CHUNKS_PER_SC + kk) * CHUNK
        coladj = [dim64 + iota + (j * 16 - ebase) for j in range(4)]
        dummy0 = iota + (CHUNK + 128 * s)
        hbase = ebase + s * TSLICE
        abase = s * TSLICE

        @pl.loop(0, NQ // 2)
        def _(g):
            o0 = g * (2 * QSLICE)
            o1 = o0 + QSLICE
            d0 = pltpu.async_copy(x_flat.at[pl.ds(hbase + o0, QSLICE)], bn0, sm0)
            d1 = pltpu.async_copy(x_flat.at[pl.ds(hbase + o1, QSLICE)], bn1, sm1)
            d0.wait()
            a0 = pltpu.async_copy(bn0, acc.at[pl.ds(abase + o0, QSLICE)], sm2)
            d1.wait()
            a1 = pltpu.async_copy(bn1, acc.at[pl.ds(abase + o1, QSLICE)], sm3)
            a0.wait()
            a1.wait()

        plsc.subcore_barrier()

        @pl.loop(0, NBATCH // 2)
        def _(g):
            b0 = s * SLAB + (2 * g) * BATCH
            b1 = b0 + BATCH
            di0 = pltpu.async_copy(idx_flat.at[pl.ds(b0, BATCH)], ibuf0, sm0)
            di1 = pltpu.async_copy(idx_flat.at[pl.ds(b1, BATCH)], ibuf1, sm1)
            di0.wait()

            @plsc.parallel_loop(0, BATCH // 64, unroll=2)
            def _(t):
                for j in range(4):
                    v = ibuf0[pl.ds(t * 64 + j * 16, 16)]
                    l = v * 64 + coladj[j]
                    ok = plsc.bitcast(l, jnp.uint32) < jnp.uint32(CHUNK)
                    dmy = dummy0 + (((t * 4 + j) & 7) * 16)
                    fbuf0[pl.ds(t * 64 + j * 16, 16)] = jnp.where(ok, l, dmy)

            sc0 = pltpu.async_copy(vals, acc.at[fbuf0], sm2, add=True)
            di1.wait()

            @plsc.parallel_loop(0, BATCH // 64, unroll=2)
            def _(t):
                for j in range(4):
                    v = ibuf1[pl.ds(t * 64 + j * 16, 16)]
                    l = v * 64 + coladj[j]
                    ok = plsc.bitcast(l, jnp.uint32) < jnp.uint32(CHUNK)
                    dmy = dummy0 + (((t * 4 + j) & 7) * 16)
                    fbuf1[pl.ds(t * 64 + j * 16, 16)] = jnp.where(ok, l, dmy)

            sc1 = pltpu.async_copy(vals, acc.at[fbuf1], sm3, add=True)
            sc0.wait()
            sc1.wait()

        plsc.subcore_barrier()

        @pl.loop(0, NQ // 2)
        def _(g):
            o0 = g * (2 * QSLICE)
            o1 = o0 + QSLICE
            d0 = pltpu.async_copy(acc.at[pl.ds(abase + o0, QSLICE)], bn0, sm0)
            d1 = pltpu.async_copy(acc.at[pl.ds(abase + o1, QSLICE)], bn1, sm1)
            d0.wait()
            a0 = pltpu.async_copy(bn0, out_flat.at[pl.ds(hbase + o0, QSLICE)], sm2)
            d1.wait()
            a1 = pltpu.async_copy(bn1, out_flat.at[pl.ds(hbase + o1, QSLICE)], sm3)
            a0.wait()
            a1.wait()


def kernel(x, dim, index, value):
    idx_flat = index.astype(jnp.int32).reshape(-1)
    vals = jnp.full((BATCH,), value, dtype=jnp.float32)
    dimarr = jnp.full((16,), dim, dtype=jnp.int32)
    out = _scatter_add(x.reshape(-1), idx_flat, vals, dimarr)
    return out.reshape(M, D)


# speedup vs baseline: 1.0794x; 1.0004x over previous
import functools

import jax
import jax.numpy as jnp
from jax import lax
from jax.experimental import pallas as pl
from jax.experimental.pallas import tpu as pltpu
from jax.experimental.pallas import tpu_sc as plsc

M, D, B = 100000, 64, 16384
NIDX = B * D
NELEM = M * D
NSC = 2
NTILE = 16
CHUNKS_PER_SC = 2
CHUNK_ROWS = M // (NSC * CHUNKS_PER_SC)
CHUNK = CHUNK_ROWS * D
PAD = 4096
SLAB = NIDX // NTILE
TSLICE = CHUNK // NTILE
QSLICE = 2000
NQ = TSLICE // QSLICE
BATCH = 4096
NBATCH = SLAB // BATCH

_mesh = plsc.VectorSubcoreMesh(core_axis_name="c", subcore_axis_name="s")


@functools.partial(
    pl.kernel,
    out_type=jax.ShapeDtypeStruct((NELEM,), jnp.float32),
    mesh=_mesh,
    scratch_types=[
        pltpu.VMEM_SHARED((CHUNK + PAD,), jnp.float32),
        pltpu.VMEM((BATCH,), jnp.int32),
        pltpu.VMEM((BATCH,), jnp.int32),
        pltpu.VMEM((BATCH,), jnp.int32),
        pltpu.VMEM((BATCH,), jnp.int32),
        pltpu.VMEM((BATCH,), jnp.float32),
        pltpu.VMEM((QSLICE,), jnp.float32),
        pltpu.VMEM((QSLICE,), jnp.float32),
        pltpu.VMEM((16,), jnp.int32),
        pltpu.SemaphoreType.DMA,
        pltpu.SemaphoreType.DMA,
        pltpu.SemaphoreType.DMA,
        pltpu.SemaphoreType.DMA,
    ],
)
def _scatter_add(x_hbm, idx_hbm, val_hbm, dim_hbm, out_hbm,
                 acc, ibuf0, ibuf1, fbuf0, fbuf1, vals, bn0, bn1, dimv,
                 sm0, sm1, sm2, sm3):
    c = lax.axis_index("c")
    s = lax.axis_index("s")
    iota = lax.iota(jnp.int32, 16)

    pltpu.sync_copy(val_hbm, vals)
    pltpu.sync_copy(dim_hbm, dimv)
    dim64 = dimv[pl.ds(0, 16)] * D

    x_flat = x_hbm
    idx_flat = idx_hbm
    out_flat = out_hbm

    for kk in range(CHUNKS_PER_SC):
        ebase = (c * CHUNKS_PER_SC + kk) * CHUNK
        coladj = [dim64 + iota + (j * 16 - ebase) for j in range(4)]
        dummy0 = iota + (CHUNK + 256 * s)
        hbase = ebase + s * TSLICE
        abase = s * TSLICE

        @pl.loop(0, NQ // 2)
        def _(g):
            o0 = g * (2 * QSLICE)
            o1 = o0 + QSLICE
            d0 = pltpu.async_copy(x_flat.at[pl.ds(hbase + o0, QSLICE)], bn0, sm0)
            d1 = pltpu.async_copy(x_flat.at[pl.ds(hbase + o1, QSLICE)], bn1, sm1)
            d0.wait()
            a0 = pltpu.async_copy(bn0, acc.at[pl.ds(abase + o0, QSLICE)], sm2)
            d1.wait()
            a1 = pltpu.async_copy(bn1, acc.at[pl.ds(abase + o1, QSLICE)], sm3)
            a0.wait()
            a1.wait()

        plsc.subcore_barrier()

        @pl.loop(0, NBATCH // 2)
        def _(g):
            b0 = s * SLAB + (2 * g) * BATCH
            b1 = b0 + BATCH
            di0 = pltpu.async_copy(idx_flat.at[pl.ds(b0, BATCH)], ibuf0, sm0)
            di1 = pltpu.async_copy(idx_flat.at[pl.ds(b1, BATCH)], ibuf1, sm1)
            di0.wait()

            @plsc.parallel_loop(0, BATCH // 64, unroll=2)
            def _(t):
                for j in range(4):
                    v = ibuf0[pl.ds(t * 64 + j * 16, 16)]
                    l = v * 64 + coladj[j]
                    ok = plsc.bitcast(l, jnp.uint32) < jnp.uint32(CHUNK)
                    dmy = dummy0 + (((t * 4 + j) & 15) * 16)
                    fbuf0[pl.ds(t * 64 + j * 16, 16)] = jnp.where(ok, l, dmy)

            sc0 = pltpu.async_copy(vals, acc.at[fbuf0], sm2, add=True)
            di1.wait()

            @plsc.parallel_loop(0, BATCH // 64, unroll=2)
            def _(t):
                for j in range(4):
                    v = ibuf1[pl.ds(t * 64 + j * 16, 16)]
                    l = v * 64 + coladj[j]
                    ok = plsc.bitcast(l, jnp.uint32) < jnp.uint32(CHUNK)
                    dmy = dummy0 + (((t * 4 + j) & 15) * 16)
                    fbuf1[pl.ds(t * 64 + j * 16, 16)] = jnp.where(ok, l, dmy)

            sc1 = pltpu.async_copy(vals, acc.at[fbuf1], sm3, add=True)
            sc0.wait()
            sc1.wait()

        plsc.subcore_barrier()

        @pl.loop(0, NQ // 2)
        def _(g):
            o0 = g * (2 * QSLICE)
            o1 = o0 + QSLICE
            d0 = pltpu.async_copy(acc.at[pl.ds(abase + o0, QSLICE)], bn0, sm0)
            d1 = pltpu.async_copy(acc.at[pl.ds(abase + o1, QSLICE)], bn1, sm1)
            d0.wait()
            a0 = pltpu.async_copy(bn0, out_flat.at[pl.ds(hbase + o0, QSLICE)], sm2)
            d1.wait()
            a1 = pltpu.async_copy(bn1, out_flat.at[pl.ds(hbase + o1, QSLICE)], sm3)
            a0.wait()
            a1.wait()


def kernel(x, dim, index, value):
    idx_flat = index.astype(jnp.int32).reshape(-1)
    vals = jnp.full((BATCH,), value, dtype=jnp.float32)
    dimarr = jnp.full((16,), dim, dtype=jnp.int32)
    out = _scatter_add(x.reshape(-1), idx_flat, vals, dimarr)
    return out.reshape(M, D)


# speedup vs baseline: 1.1748x; 1.0884x over previous
import functools

import jax
import jax.numpy as jnp
from jax import lax
from jax.experimental import pallas as pl
from jax.experimental.pallas import tpu as pltpu
from jax.experimental.pallas import tpu_sc as plsc

M, D, B = 100000, 64, 16384
NIDX = B * D
NELEM = M * D
NSC = 2
NTILE = 16
CHUNKS_PER_SC = 2
CHUNK_ROWS = M // (NSC * CHUNKS_PER_SC)
CHUNK = CHUNK_ROWS * D
PAD = 2048
SLAB = NIDX // NTILE
TSLICE = CHUNK // NTILE
QSLICE = 5000
NQ = TSLICE // QSLICE
BATCH = 4096
NBATCH = SLAB // BATCH

_mesh = plsc.VectorSubcoreMesh(core_axis_name="c", subcore_axis_name="s")


@functools.partial(
    pl.kernel,
    out_type=jax.ShapeDtypeStruct((NELEM,), jnp.float32),
    mesh=_mesh,
    scratch_types=[
        pltpu.VMEM_SHARED((CHUNK + PAD,), jnp.float32),
        pltpu.VMEM((BATCH,), jnp.int32),
        pltpu.VMEM((BATCH,), jnp.int32),
        pltpu.VMEM((BATCH,), jnp.int32),
        pltpu.VMEM((BATCH,), jnp.int32),
        pltpu.VMEM((BATCH,), jnp.float32),
        pltpu.VMEM((QSLICE,), jnp.float32),
        pltpu.VMEM((QSLICE,), jnp.float32),
        pltpu.VMEM((16,), jnp.int32),
        pltpu.SemaphoreType.DMA,
        pltpu.SemaphoreType.DMA,
        pltpu.SemaphoreType.DMA,
        pltpu.SemaphoreType.DMA,
    ],
)
def _scatter_add(x_hbm, idx_hbm, val_hbm, dim_hbm, out_hbm,
                 acc, ibuf0, ibuf1, fbuf0, fbuf1, vals, bn0, bn1, dimv,
                 sm0, sm1, sm2, sm3):
    c = lax.axis_index("c")
    s = lax.axis_index("s")
    iota = lax.iota(jnp.int32, 16)

    pltpu.sync_copy(val_hbm, vals)
    pltpu.sync_copy(dim_hbm, dimv)
    dim64 = dimv[pl.ds(0, 16)] * D

    x_flat = x_hbm
    idx_flat = idx_hbm
    out_flat = out_hbm

    for kk in range(CHUNKS_PER_SC):
        ebase = (c * CHUNKS_PER_SC + kk) * CHUNK
        coladj = [dim64 + iota + (j * 16 - ebase) for j in range(4)]
        dummy0 = iota + (CHUNK + 128 * s)
        hbase = ebase + s * TSLICE
        abase = s * TSLICE

        @pl.loop(0, NQ // 2)
        def _(g):
            o0 = g * (2 * QSLICE)
            o1 = o0 + QSLICE
            d0 = pltpu.async_copy(x_flat.at[pl.ds(hbase + o0, QSLICE)], bn0, sm0)
            d1 = pltpu.async_copy(x_flat.at[pl.ds(hbase + o1, QSLICE)], bn1, sm1)
            d0.wait()
            a0 = pltpu.async_copy(bn0, acc.at[pl.ds(abase + o0, QSLICE)], sm2)
            d1.wait()
            a1 = pltpu.async_copy(bn1, acc.at[pl.ds(abase + o1, QSLICE)], sm3)
            a0.wait()
            a1.wait()

        plsc.subcore_barrier()

        @pl.loop(0, NBATCH // 2)
        def _(g):
            b0 = s * SLAB + (2 * g) * BATCH
            b1 = b0 + BATCH
            di0 = pltpu.async_copy(idx_flat.at[pl.ds(b0, BATCH)], ibuf0, sm0)
            di1 = pltpu.async_copy(idx_flat.at[pl.ds(b1, BATCH)], ibuf1, sm1)
            di0.wait()

            @plsc.parallel_loop(0, BATCH // 64, unroll=2)
            def _(t):
                for j in range(4):
                    v = ibuf0[pl.ds(t * 64 + j * 16, 16)]
                    l = v * 64 + coladj[j]
                    ok = plsc.bitcast(l, jnp.uint32) < jnp.uint32(CHUNK)
                    dmy = dummy0 + (((t * 4 + j) & 7) * 16)
                    fbuf0[pl.ds(t * 64 + j * 16, 16)] = jnp.where(ok, l, dmy)

            sc0 = pltpu.async_copy(vals, acc.at[fbuf0], sm2, add=True)
            di1.wait()

            @plsc.parallel_loop(0, BATCH // 64, unroll=2)
            def _(t):
                for j in range(4):
                    v = ibuf1[pl.ds(t * 64 + j * 16, 16)]
                    l = v * 64 + coladj[j]
                    ok = plsc.bitcast(l, jnp.uint32) < jnp.uint32(CHUNK)
                    dmy = dummy0 + (((t * 4 + j) & 7) * 16)
                    fbuf1[pl.ds(t * 64 + j * 16, 16)] = jnp.where(ok, l, dmy)

            sc1 = pltpu.async_copy(vals, acc.at[fbuf1], sm3, add=True)
            sc0.wait()
            sc1.wait()

        plsc.subcore_barrier()

        @pl.loop(0, NQ // 2)
        def _(g):
            o0 = g * (2 * QSLICE)
            o1 = o0 + QSLICE
            d0 = pltpu.async_copy(acc.at[pl.ds(abase + o0, QSLICE)], bn0, sm0)
            d1 = pltpu.async_copy(acc.at[pl.ds(abase + o1, QSLICE)], bn1, sm1)
            d0.wait()
            a0 = pltpu.async_copy(bn0, out_flat.at[pl.ds(hbase + o0, QSLICE)], sm2)
            d1.wait()
            a1 = pltpu.async_copy(bn1, out_flat.at[pl.ds(hbase + o1, QSLICE)], sm3)
            a0.wait()
            a1.wait()


def kernel(x, dim, index, value):
    idx_flat = index.astype(jnp.int32).reshape(-1)
    vals = jnp.full((BATCH,), value, dtype=jnp.float32)
    dimarr = jnp.full((16,), dim, dtype=jnp.int32)
    out = _scatter_add(x.reshape(-1), idx_flat, vals, dimarr)
    return out.reshape(M, D)
